# Initial kernel scaffold; baseline (speedup 1.0000x reference)
import jax
import jax.numpy as jnp
from jax.experimental import pallas as pl


def kernel(e_sender_indices, e_receiver_indices, r_sender_indices, r_receiver_indices, E_forward, E_backward, R_forward, R_backward, E_message_b, E_gate_b, R_message_b, R_gate_b):
    raise NotImplementedError("write your pallas kernel here")


# speedup vs baseline: 2.4226x; 2.4226x over previous
import functools

import jax
import jax.numpy as jnp
from jax import lax
from jax.experimental import pallas as pl
from jax.experimental.pallas import tpu as pltpu
from jax.experimental.pallas import tpu_sc as plsc

NENT = 10000
NREL = 10000
NEDGE = 320000
D = 128
HALF = D // 2
NC = 2
NS = 16
B = 128
NBLK = 160
EPT = NBLK * B
EPS = EPT * NS
ROWS = 10112
RPT = ROWS // NS
NCHUNK = RPT // 8
TRASH = NENT


def _sc_body(m_lo_ref, m_hi_ref, gidx_ref, sidx_ref, zacc_ref,
             s_out_ref,
             deg_sh, acc_sh,
             gidx_b, sidx_b, rows_v, ones_v, zero8_v, stage_v, dstage_v, sem):
    cid = lax.axis_index("c")
    sid = lax.axis_index("s")
    r0 = sid * RPT

    ones16 = jnp.full((16,), 1.0, jnp.float32)
    zero16 = jnp.zeros((16,), jnp.float32)

    def fill_ones(r, carry):
        ones_v[r, :] = ones16
        return carry

    lax.fori_loop(0, B, fill_ones, 0)
    for r in range(8):
        zero8_v[r, :] = zero16

    for phase in range(2):
        sum_id = phase * 2 + cid
        base = (sum_id * NS + sid) * EPT
        for h in range(2):
            m_ref = (m_lo_ref, m_hi_ref)[h]
            pltpu.sync_copy(zacc_ref.at[pl.ds(r0, RPT)],
                            acc_sh.at[pl.ds(r0, RPT)])
            if h == 0:
                def zero_deg(c, carry):
                    pltpu.sync_copy(zero8_v, deg_sh.at[pl.ds(r0 + c * 8, 8)])
                    return carry

                lax.fori_loop(0, NCHUNK, zero_deg, 0)
            plsc.subcore_barrier()

            def blk(j, carry):
                off = base + j * B
                pltpu.sync_copy(gidx_ref.at[pl.ds(off, B)], gidx_b)
                pltpu.sync_copy(sidx_ref.at[pl.ds(off, B)], sidx_b)
                pltpu.async_copy(m_ref.at[gidx_b], rows_v, sem).wait()
                pltpu.sync_copy(rows_v, acc_sh.at[sidx_b], add=True)
                if h == 0:
                    pltpu.sync_copy(ones_v, deg_sh.at[sidx_b], add=True)
                return carry

            lax.fori_loop(0, NBLK, blk, 0)
            plsc.subcore_barrier()

            def norm_chunk(c, carry):
                r = r0 + c * 8
                pltpu.sync_copy(acc_sh.at[pl.ds(r, 8)], stage_v)
                pltpu.sync_copy(deg_sh.at[pl.ds(r, 8)], dstage_v)
                for rr in range(8):
                    inv = 1.0 / jnp.maximum(dstage_v[rr, :], 1.0)
                    for k in range(HALF // 16):
                        sl = pl.ds(k * 16, 16)
                        stage_v[rr, sl] = stage_v[rr, sl] * inv
                pltpu.sync_copy(stage_v, s_out_ref.at[sum_id, h, pl.ds(r, 8)])
                return carry

            lax.fori_loop(0, NCHUNK, norm_chunk, 0)


@functools.lru_cache(maxsize=1)
def _sc_collect():
    mesh = plsc.VectorSubcoreMesh(core_axis_name="c", subcore_axis_name="s",
                                  num_cores=NC, num_subcores=NS)
    return pl.kernel(
        _sc_body,
        out_type=jax.ShapeDtypeStruct((4, 2, ROWS, HALF), jnp.float32),
        mesh=mesh,
        compiler_params=pltpu.CompilerParams(use_tc_tiling_on_sc=False),
        scratch_types=[
            pltpu.VMEM_SHARED((ROWS, 16), jnp.float32),
            pltpu.VMEM_SHARED((ROWS, HALF), jnp.float32),
            pltpu.VMEM((B,), jnp.int32),
            pltpu.VMEM((B,), jnp.int32),
            pltpu.VMEM((B, HALF), jnp.float32),
            pltpu.VMEM((B, 16), jnp.float32),
            pltpu.VMEM((8, 16), jnp.float32),
            pltpu.VMEM((8, HALF), jnp.float32),
            pltpu.VMEM((8, 16), jnp.float32),
            pltpu.SemaphoreType.DMA,
        ],
    )


def _relu_body(t_ref, b_ref, lo_ref, hi_ref):
    res = jnp.maximum(t_ref[...] + b_ref[...], 0.0)
    lo_ref[...] = res[..., :HALF]
    hi_ref[...] = res[..., HALF:]


def _relu_tables(tables, biases):
    rb = 1000
    return pl.pallas_call(
        _relu_body,
        grid=(4, NENT // rb),
        in_specs=[pl.BlockSpec((1, rb, D), lambda j, i: (j, i, 0)),
                  pl.BlockSpec((1, 1, D), lambda j, i: (j, 0, 0))],
        out_specs=[pl.BlockSpec((1, rb, HALF), lambda j, i: (j, i, 0)),
                   pl.BlockSpec((1, rb, HALF), lambda j, i: (j, i, 0))],
        out_shape=[jax.ShapeDtypeStruct((4, NENT, HALF), jnp.float32),
                   jax.ShapeDtypeStruct((4, NENT, HALF), jnp.float32)],
    )(tables, biases)


def _sum_body(s_ref, oe_ref, or_ref):
    s = s_ref[...]
    oe_ref[...] = jnp.concatenate([s[0, 0] + s[1, 0], s[0, 1] + s[1, 1]],
                                  axis=-1)
    or_ref[...] = jnp.concatenate([s[2, 0] + s[3, 0], s[2, 1] + s[3, 1]],
                                  axis=-1)


def _combine(s_norm):
    rb = 1000
    return pl.pallas_call(
        _sum_body,
        grid=(NENT // rb,),
        in_specs=[pl.BlockSpec((4, 2, rb, HALF), lambda i: (0, 0, i, 0))],
        out_specs=[pl.BlockSpec((rb, D), lambda i: (i, 0)),
                   pl.BlockSpec((rb, D), lambda i: (i, 0))],
        out_shape=[jax.ShapeDtypeStruct((NENT, D), jnp.float32),
                   jax.ShapeDtypeStruct((NREL, D), jnp.float32)],
    )(s_norm)


def kernel(e_sender_indices, e_receiver_indices, r_sender_indices,
           r_receiver_indices, E_forward, E_backward, R_forward, R_backward,
           E_message_b, E_gate_b, R_message_b, R_gate_b):
    tables = jnp.stack([R_forward, R_backward, E_forward, E_backward])
    biases = jnp.stack([R_message_b, R_gate_b, E_message_b, E_gate_b])[:, None, :]
    m_lo, m_hi = _relu_tables(tables, biases)
    m_lo = m_lo.reshape(4 * NENT, HALF)
    m_hi = m_hi.reshape(4 * NENT, HALF)

    gidx = jnp.stack([r_sender_indices,
                      r_receiver_indices + NENT,
                      e_sender_indices + 2 * NENT,
                      e_receiver_indices + 3 * NENT])
    sidx = jnp.stack([e_receiver_indices, e_sender_indices,
                      r_receiver_indices, r_sender_indices])
    pad = EPS - NEDGE
    gidx = jnp.pad(gidx, ((0, 0), (0, pad))).reshape(4 * NS * EPT)
    sidx = jnp.pad(sidx, ((0, 0), (0, pad)),
                   constant_values=TRASH).reshape(4 * NS * EPT)

    s_norm = _sc_collect()(m_lo, m_hi, gidx, sidx,
                           jnp.zeros((ROWS, HALF), jnp.float32))

    return _combine(s_norm)


# speedup vs baseline: 3.4269x; 1.4145x over previous
import functools

import jax
import jax.numpy as jnp
from jax import lax
from jax.experimental import pallas as pl
from jax.experimental.pallas import tpu as pltpu
from jax.experimental.pallas import tpu_sc as plsc

NENT = 10000
NREL = 10000
NEDGE = 320000
D = 128
HALF = D // 2
NC = 2
NS = 16
B = 128
NBLK = 160
EPT = NBLK * B
EPS = EPT * NS
ROWS = 10112
RPT = ROWS // NS
NCHUNK = RPT // 8
TRASH = NENT


def _sc_body(m_lo_ref, m_hi_ref, gidx_ref, sidx_ref, zacc_ref,
             s_out_ref,
             deg_sh, acc_sh,
             gidx_t, sidx_t, buf0, buf1, ones_v, zero8_v, stage_v, dstage_v,
             sem0, sem1):
    cid = lax.axis_index("c")
    sid = lax.axis_index("s")
    r0 = sid * RPT

    ones16 = jnp.full((16,), 1.0, jnp.float32)
    zero16 = jnp.zeros((16,), jnp.float32)

    def fill_ones(r, carry):
        ones_v[r, :] = ones16
        return carry

    lax.fori_loop(0, B, fill_ones, 0)
    for r in range(8):
        zero8_v[r, :] = zero16

    for phase in range(2):
        sum_id = phase * 2 + cid
        w = sum_id * NS + sid
        pltpu.sync_copy(gidx_ref.at[w], gidx_t)
        pltpu.sync_copy(sidx_ref.at[w], sidx_t)
        for h in range(2):
            m_ref = (m_lo_ref, m_hi_ref)[h]
            pltpu.sync_copy(zacc_ref.at[pl.ds(r0, RPT)],
                            acc_sh.at[pl.ds(r0, RPT)])
            if h == 0:
                def zero_deg(c, carry):
                    pltpu.sync_copy(zero8_v, deg_sh.at[pl.ds(r0 + c * 8, 8)])
                    return carry

                lax.fori_loop(0, NCHUNK, zero_deg, 0)
            plsc.subcore_barrier()

            pltpu.async_copy(m_ref.at[gidx_t.at[0]], buf0, sem0)

            def blk2(i, carry):
                j0 = 2 * i
                pltpu.async_copy(m_ref.at[gidx_t.at[j0 + 1]], buf1, sem1)
                pltpu.make_async_copy(m_ref.at[gidx_t.at[j0]], buf0,
                                      sem0).wait()
                pltpu.sync_copy(buf0, acc_sh.at[sidx_t.at[j0]], add=True)
                if h == 0:
                    pltpu.sync_copy(ones_v, deg_sh.at[sidx_t.at[j0]],
                                    add=True)

                @pl.when(j0 + 2 < NBLK)
                def _():
                    pltpu.async_copy(m_ref.at[gidx_t.at[j0 + 2]], buf0, sem0)

                pltpu.make_async_copy(m_ref.at[gidx_t.at[j0 + 1]], buf1,
                                      sem1).wait()
                pltpu.sync_copy(buf1, acc_sh.at[sidx_t.at[j0 + 1]], add=True)
                if h == 0:
                    pltpu.sync_copy(ones_v, deg_sh.at[sidx_t.at[j0 + 1]],
                                    add=True)
                return carry

            lax.fori_loop(0, NBLK // 2, blk2, 0)
            plsc.subcore_barrier()

            def norm_chunk(c, carry):
                r = r0 + c * 8
                pltpu.sync_copy(acc_sh.at[pl.ds(r, 8)], stage_v)
                pltpu.sync_copy(deg_sh.at[pl.ds(r, 8)], dstage_v)
                for rr in range(8):
                    inv = 1.0 / jnp.maximum(dstage_v[rr, :], 1.0)
                    for k in range(HALF // 16):
                        sl = pl.ds(k * 16, 16)
                        stage_v[rr, sl] = stage_v[rr, sl] * inv
                pltpu.sync_copy(stage_v, s_out_ref.at[sum_id, h, pl.ds(r, 8)])
                return carry

            lax.fori_loop(0, NCHUNK, norm_chunk, 0)


@functools.lru_cache(maxsize=1)
def _sc_collect():
    mesh = plsc.VectorSubcoreMesh(core_axis_name="c", subcore_axis_name="s",
                                  num_cores=NC, num_subcores=NS)
    return pl.kernel(
        _sc_body,
        out_type=jax.ShapeDtypeStruct((4, 2, ROWS, HALF), jnp.float32),
        mesh=mesh,
        compiler_params=pltpu.CompilerParams(use_tc_tiling_on_sc=False),
        scratch_types=[
            pltpu.VMEM_SHARED((ROWS, 16), jnp.float32),
            pltpu.VMEM_SHARED((ROWS, HALF), jnp.float32),
            pltpu.VMEM((NBLK, B), jnp.int32),
            pltpu.VMEM((NBLK, B), jnp.int32),
            pltpu.VMEM((B, HALF), jnp.float32),
            pltpu.VMEM((B, HALF), jnp.float32),
            pltpu.VMEM((B, 16), jnp.float32),
            pltpu.VMEM((8, 16), jnp.float32),
            pltpu.VMEM((8, HALF), jnp.float32),
            pltpu.VMEM((8, 16), jnp.float32),
            pltpu.SemaphoreType.DMA,
            pltpu.SemaphoreType.DMA,
        ],
    )


def _relu_body(t_ref, b_ref, lo_ref, hi_ref):
    res = jnp.maximum(t_ref[...] + b_ref[...], 0.0)
    lo_ref[...] = res[..., :HALF]
    hi_ref[...] = res[..., HALF:]


def _relu_tables(tables, biases):
    rb = 1000
    return pl.pallas_call(
        _relu_body,
        grid=(4, NENT // rb),
        in_specs=[pl.BlockSpec((1, rb, D), lambda j, i: (j, i, 0)),
                  pl.BlockSpec((1, 1, D), lambda j, i: (j, 0, 0))],
        out_specs=[pl.BlockSpec((1, rb, HALF), lambda j, i: (j, i, 0)),
                   pl.BlockSpec((1, rb, HALF), lambda j, i: (j, i, 0))],
        out_shape=[jax.ShapeDtypeStruct((4, NENT, HALF), jnp.float32),
                   jax.ShapeDtypeStruct((4, NENT, HALF), jnp.float32)],
    )(tables, biases)


def _sum_body(s_ref, oe_ref, or_ref):
    s = s_ref[...]
    oe_ref[...] = jnp.concatenate([s[0, 0] + s[1, 0], s[0, 1] + s[1, 1]],
                                  axis=-1)
    or_ref[...] = jnp.concatenate([s[2, 0] + s[3, 0], s[2, 1] + s[3, 1]],
                                  axis=-1)


def _combine(s_norm):
    rb = 1000
    return pl.pallas_call(
        _sum_body,
        grid=(NENT // rb,),
        in_specs=[pl.BlockSpec((4, 2, rb, HALF), lambda i: (0, 0, i, 0))],
        out_specs=[pl.BlockSpec((rb, D), lambda i: (i, 0)),
                   pl.BlockSpec((rb, D), lambda i: (i, 0))],
        out_shape=[jax.ShapeDtypeStruct((NENT, D), jnp.float32),
                   jax.ShapeDtypeStruct((NREL, D), jnp.float32)],
    )(s_norm)


def kernel(e_sender_indices, e_receiver_indices, r_sender_indices,
           r_receiver_indices, E_forward, E_backward, R_forward, R_backward,
           E_message_b, E_gate_b, R_message_b, R_gate_b):
    tables = jnp.stack([R_forward, R_backward, E_forward, E_backward])
    biases = jnp.stack([R_message_b, R_gate_b, E_message_b, E_gate_b])[:, None, :]
    m_lo, m_hi = _relu_tables(tables, biases)
    m_lo = m_lo.reshape(4 * NENT, HALF)
    m_hi = m_hi.reshape(4 * NENT, HALF)

    gidx = jnp.stack([r_sender_indices,
                      r_receiver_indices + NENT,
                      e_sender_indices + 2 * NENT,
                      e_receiver_indices + 3 * NENT])
    sidx = jnp.stack([e_receiver_indices, e_sender_indices,
                      r_receiver_indices, r_sender_indices])
    pad = EPS - NEDGE
    gidx = jnp.pad(gidx, ((0, 0), (0, pad))).reshape(4 * NS, NBLK, B)
    sidx = jnp.pad(sidx, ((0, 0), (0, pad)),
                   constant_values=TRASH).reshape(4 * NS, NBLK, B)

    s_norm = _sc_collect()(m_lo, m_hi, gidx, sidx,
                           jnp.zeros((ROWS, HALF), jnp.float32))

    return _combine(s_norm)


# speedup vs baseline: 3.5776x; 1.0440x over previous
import functools

import jax
import jax.numpy as jnp
from jax import lax
from jax.experimental import pallas as pl
from jax.experimental.pallas import tpu as pltpu
from jax.experimental.pallas import tpu_sc as plsc

NENT = 10000
NREL = 10000
NEDGE = 320000
D = 128
HALF = D // 2
NC = 2
NS = 16
B = 128
NBLK = 160
EPT = NBLK * B
EPS = EPT * NS
ROWS = 10112
RPT = ROWS // NS
NCHUNK = RPT // 8
TRASH = NENT


def _sc_body(m_lo_ref, m_hi_ref, gidx_ref, sidx_ref, zacc_ref,
             s_out_ref,
             deg_sh, acc_sh,
             gidx_t, sidx_t, buf0, buf1, buf2, buf3,
             ones_v, zero8_v, stage_v, dstage_v,
             gsem0, gsem1, gsem2, gsem3, ssem0, ssem1, ssem2, ssem3, dsem):
    bufs = (buf0, buf1, buf2, buf3)
    gsems = (gsem0, gsem1, gsem2, gsem3)
    ssems = (ssem0, ssem1, ssem2, ssem3)
    cid = lax.axis_index("c")
    sid = lax.axis_index("s")
    r0 = sid * RPT

    ones16 = jnp.full((16,), 1.0, jnp.float32)
    zero16 = jnp.zeros((16,), jnp.float32)

    def fill_ones(r, carry):
        ones_v[r, :] = ones16
        return carry

    lax.fori_loop(0, B, fill_ones, 0)
    for r in range(8):
        zero8_v[r, :] = zero16

    for phase in range(2):
        sum_id = phase * 2 + cid
        w = sum_id * NS + sid
        pltpu.sync_copy(gidx_ref.at[w], gidx_t)
        pltpu.sync_copy(sidx_ref.at[w], sidx_t)
        for h in range(2):
            m_ref = (m_lo_ref, m_hi_ref)[h]
            pltpu.sync_copy(zacc_ref.at[pl.ds(r0, RPT)],
                            acc_sh.at[pl.ds(r0, RPT)])
            if h == 0:
                def zero_deg(c, carry):
                    pltpu.sync_copy(zero8_v, deg_sh.at[pl.ds(r0 + c * 8, 8)])
                    return carry

                lax.fori_loop(0, NCHUNK, zero_deg, 0)
            plsc.subcore_barrier()

            G = len(bufs)
            for b in range(G):
                pltpu.async_copy(m_ref.at[gidx_t.at[b]], bufs[b], gsems[b])

            def ring(i, carry):
                for b in range(G):
                    j = G * i + b
                    pltpu.make_async_copy(m_ref.at[gidx_t.at[j]], bufs[b],
                                          gsems[b]).wait()
                    pltpu.make_async_copy(bufs[b], acc_sh.at[sidx_t.at[j]],
                                          ssems[b]).start(add=True)
                    if h == 0:
                        pltpu.make_async_copy(ones_v,
                                              deg_sh.at[sidx_t.at[j]],
                                              dsem).start(add=True)

                    @pl.when(j + G < NBLK)
                    def _():
                        pltpu.make_async_copy(bufs[b],
                                              acc_sh.at[sidx_t.at[j]],
                                              ssems[b]).wait()
                        pltpu.async_copy(m_ref.at[gidx_t.at[j + G]], bufs[b],
                                         gsems[b])
                return carry

            lax.fori_loop(0, NBLK // G, ring, 0)
            for b in range(G):
                pltpu.make_async_copy(bufs[b],
                                      acc_sh.at[sidx_t.at[NBLK - G + b]],
                                      ssems[b]).wait()
            if h == 0:
                def drain_deg(i, carry):
                    pltpu.make_async_copy(ones_v, deg_sh.at[sidx_t.at[0]],
                                          dsem).wait()
                    return carry

                lax.fori_loop(0, NBLK, drain_deg, 0)
            plsc.subcore_barrier()

            def norm_chunk(c, carry):
                r = r0 + c * 8
                pltpu.sync_copy(acc_sh.at[pl.ds(r, 8)], stage_v)
                pltpu.sync_copy(deg_sh.at[pl.ds(r, 8)], dstage_v)
                for rr in range(8):
                    inv = 1.0 / jnp.maximum(dstage_v[rr, :], 1.0)
                    for k in range(HALF // 16):
                        sl = pl.ds(k * 16, 16)
                        stage_v[rr, sl] = stage_v[rr, sl] * inv
                pltpu.sync_copy(stage_v, s_out_ref.at[sum_id, h, pl.ds(r, 8)])
                return carry

            lax.fori_loop(0, NCHUNK, norm_chunk, 0)


@functools.lru_cache(maxsize=1)
def _sc_collect():
    mesh = plsc.VectorSubcoreMesh(core_axis_name="c", subcore_axis_name="s",
                                  num_cores=NC, num_subcores=NS)
    return pl.kernel(
        _sc_body,
        out_type=jax.ShapeDtypeStruct((4, 2, ROWS, HALF), jnp.float32),
        mesh=mesh,
        compiler_params=pltpu.CompilerParams(use_tc_tiling_on_sc=False),
        scratch_types=[
            pltpu.VMEM_SHARED((ROWS, 16), jnp.float32),
            pltpu.VMEM_SHARED((ROWS, HALF), jnp.float32),
            pltpu.VMEM((NBLK, B), jnp.int32),
            pltpu.VMEM((NBLK, B), jnp.int32),
            pltpu.VMEM((B, HALF), jnp.float32),
            pltpu.VMEM((B, HALF), jnp.float32),
            pltpu.VMEM((B, HALF), jnp.float32),
            pltpu.VMEM((B, HALF), jnp.float32),
            pltpu.VMEM((B, 16), jnp.float32),
            pltpu.VMEM((8, 16), jnp.float32),
            pltpu.VMEM((8, HALF), jnp.float32),
            pltpu.VMEM((8, 16), jnp.float32),
        ] + [pltpu.SemaphoreType.DMA] * 9,
    )


def _relu_body(t_ref, b_ref, lo_ref, hi_ref):
    res = jnp.maximum(t_ref[...] + b_ref[...], 0.0)
    lo_ref[...] = res[..., :HALF]
    hi_ref[...] = res[..., HALF:]


def _relu_tables(tables, biases):
    rb = 1000
    return pl.pallas_call(
        _relu_body,
        grid=(4, NENT // rb),
        in_specs=[pl.BlockSpec((1, rb, D), lambda j, i: (j, i, 0)),
                  pl.BlockSpec((1, 1, D), lambda j, i: (j, 0, 0))],
        out_specs=[pl.BlockSpec((1, rb, HALF), lambda j, i: (j, i, 0)),
                   pl.BlockSpec((1, rb, HALF), lambda j, i: (j, i, 0))],
        out_shape=[jax.ShapeDtypeStruct((4, NENT, HALF), jnp.float32),
                   jax.ShapeDtypeStruct((4, NENT, HALF), jnp.float32)],
    )(tables, biases)


def _sum_body(s_ref, oe_ref, or_ref):
    s = s_ref[...]
    oe_ref[...] = jnp.concatenate([s[0, 0] + s[1, 0], s[0, 1] + s[1, 1]],
                                  axis=-1)
    or_ref[...] = jnp.concatenate([s[2, 0] + s[3, 0], s[2, 1] + s[3, 1]],
                                  axis=-1)


def _combine(s_norm):
    rb = 1000
    return pl.pallas_call(
        _sum_body,
        grid=(NENT // rb,),
        in_specs=[pl.BlockSpec((4, 2, rb, HALF), lambda i: (0, 0, i, 0))],
        out_specs=[pl.BlockSpec((rb, D), lambda i: (i, 0)),
                   pl.BlockSpec((rb, D), lambda i: (i, 0))],
        out_shape=[jax.ShapeDtypeStruct((NENT, D), jnp.float32),
                   jax.ShapeDtypeStruct((NREL, D), jnp.float32)],
    )(s_norm)


def kernel(e_sender_indices, e_receiver_indices, r_sender_indices,
           r_receiver_indices, E_forward, E_backward, R_forward, R_backward,
           E_message_b, E_gate_b, R_message_b, R_gate_b):
    tables = jnp.stack([R_forward, R_backward, E_forward, E_backward])
    biases = jnp.stack([R_message_b, R_gate_b, E_message_b, E_gate_b])[:, None, :]
    m_lo, m_hi = _relu_tables(tables, biases)
    m_lo = m_lo.reshape(4 * NENT, HALF)
    m_hi = m_hi.reshape(4 * NENT, HALF)

    gidx = jnp.stack([r_sender_indices,
                      r_receiver_indices + NENT,
                      e_sender_indices + 2 * NENT,
                      e_receiver_indices + 3 * NENT])
    sidx = jnp.stack([e_receiver_indices, e_sender_indices,
                      r_receiver_indices, r_sender_indices])
    pad = EPS - NEDGE
    gidx = jnp.pad(gidx, ((0, 0), (0, pad))).reshape(4 * NS, NBLK, B)
    sidx = jnp.pad(sidx, ((0, 0), (0, pad)),
                   constant_values=TRASH).reshape(4 * NS, NBLK, B)

    s_norm = _sc_collect()(m_lo, m_hi, gidx, sidx,
                           jnp.zeros((ROWS, HALF), jnp.float32))

    return _combine(s_norm)


# speedup vs baseline: 6.3908x; 1.7863x over previous
import functools

import jax
import jax.numpy as jnp
from jax import lax
from jax.experimental import pallas as pl
from jax.experimental.pallas import tpu as pltpu
from jax.experimental.pallas import tpu_sc as plsc

NENT = 10000
NREL = 10000
NEDGE = 320000
D = 128
NC = 2
NS = 16
B = 128
NBLK = 160
EPT = NBLK * B
EPS = EPT * NS
ROWS = 10112
RPT = ROWS // NS
NCHUNK = RPT // 8
TRASH = NENT


def _sc_body(m_ref, gidx_ref, sidx_ref, zacc_ref,
             s_out_ref, deg_out_ref,
             deg_sh, acc_sh,
             gidx_t, sidx_t, buf0, buf1, buf2, buf3,
             ones_v, zero8_v,
             gsem0, gsem1, gsem2, gsem3, ssem0, ssem1, ssem2, ssem3, dsem):
    bufs = (buf0, buf1, buf2, buf3)
    gsems = (gsem0, gsem1, gsem2, gsem3)
    ssems = (ssem0, ssem1, ssem2, ssem3)
    cid = lax.axis_index("c")
    sid = lax.axis_index("s")
    r0 = sid * RPT

    ones16 = jnp.full((16,), 1.0, jnp.float32)
    zero16 = jnp.zeros((16,), jnp.float32)

    def fill_ones(r, carry):
        ones_v[r, :] = ones16
        return carry

    lax.fori_loop(0, B, fill_ones, 0)
    for r in range(8):
        zero8_v[r, :] = zero16

    for phase in range(2):
        sum_id = phase * 2 + cid
        w = sum_id * NS + sid
        pltpu.sync_copy(gidx_ref.at[w], gidx_t)
        pltpu.sync_copy(sidx_ref.at[w], sidx_t)
        pltpu.sync_copy(zacc_ref.at[pl.ds(r0, RPT)], acc_sh.at[pl.ds(r0, RPT)])

        def zero_deg(c, carry):
            pltpu.sync_copy(zero8_v, deg_sh.at[pl.ds(r0 + c * 8, 8)])
            return carry

        lax.fori_loop(0, NCHUNK, zero_deg, 0)
        plsc.subcore_barrier()

        G = len(bufs)
        for b in range(G):
            pltpu.async_copy(m_ref.at[gidx_t.at[b]], bufs[b], gsems[b])

        def ring(i, carry):
            for b in range(G):
                j = G * i + b
                pltpu.make_async_copy(m_ref.at[gidx_t.at[j]], bufs[b],
                                      gsems[b]).wait()
                pltpu.make_async_copy(bufs[b], acc_sh.at[sidx_t.at[j]],
                                      ssems[b]).start(add=True)
                pltpu.make_async_copy(ones_v, deg_sh.at[sidx_t.at[j]],
                                      dsem).start(add=True)

                @pl.when(j + G < NBLK)
                def _():
                    pltpu.make_async_copy(bufs[b], acc_sh.at[sidx_t.at[j]],
                                          ssems[b]).wait()
                    pltpu.async_copy(m_ref.at[gidx_t.at[j + G]], bufs[b],
                                     gsems[b])
            return carry

        lax.fori_loop(0, NBLK // G, ring, 0)
        for b in range(G):
            pltpu.make_async_copy(bufs[b],
                                  acc_sh.at[sidx_t.at[NBLK - G + b]],
                                  ssems[b]).wait()

        def drain_deg(i, carry):
            pltpu.make_async_copy(ones_v, deg_sh.at[sidx_t.at[0]],
                                  dsem).wait()
            return carry

        lax.fori_loop(0, NBLK, drain_deg, 0)
        plsc.subcore_barrier()

        pltpu.sync_copy(acc_sh.at[pl.ds(r0, RPT)],
                        s_out_ref.at[sum_id, pl.ds(r0, RPT)])
        pltpu.sync_copy(deg_sh.at[pl.ds(r0, RPT)],
                        deg_out_ref.at[sum_id, pl.ds(r0, RPT)])


@functools.lru_cache(maxsize=1)
def _sc_collect():
    mesh = plsc.VectorSubcoreMesh(core_axis_name="c", subcore_axis_name="s",
                                  num_cores=NC, num_subcores=NS)
    return pl.kernel(
        _sc_body,
        out_type=[jax.ShapeDtypeStruct((4, ROWS, D), jnp.bfloat16),
                  jax.ShapeDtypeStruct((4, ROWS, 16), jnp.float32)],
        mesh=mesh,
        compiler_params=pltpu.CompilerParams(use_tc_tiling_on_sc=False),
        scratch_types=[
            pltpu.VMEM_SHARED((ROWS, 16), jnp.float32),
            pltpu.VMEM_SHARED((ROWS, D), jnp.bfloat16),
            pltpu.VMEM((NBLK, B), jnp.int32),
            pltpu.VMEM((NBLK, B), jnp.int32),
            pltpu.VMEM((B, D), jnp.bfloat16),
            pltpu.VMEM((B, D), jnp.bfloat16),
            pltpu.VMEM((B, D), jnp.bfloat16),
            pltpu.VMEM((B, D), jnp.bfloat16),
            pltpu.VMEM((B, 16), jnp.float32),
            pltpu.VMEM((8, 16), jnp.float32),
        ] + [pltpu.SemaphoreType.DMA] * 9,
    )


def _relu_body(t_ref, b_ref, o_ref):
    o_ref[...] = jnp.maximum(t_ref[...] + b_ref[...], 0.0).astype(jnp.bfloat16)


def _relu_tables(tables, biases):
    rb = 1000
    return pl.pallas_call(
        _relu_body,
        grid=(4, NENT // rb),
        in_specs=[pl.BlockSpec((1, rb, D), lambda j, i: (j, i, 0)),
                  pl.BlockSpec((1, 1, D), lambda j, i: (j, 0, 0))],
        out_specs=pl.BlockSpec((1, rb, D), lambda j, i: (j, i, 0)),
        out_shape=jax.ShapeDtypeStruct((4, NENT, D), jnp.bfloat16),
    )(tables, biases)


def _norm_body(s_ref, d_ref, oe_ref, or_ref):
    s = s_ref[...].astype(jnp.float32)
    d = jnp.maximum(d_ref[...][:, :, 0:1], 1.0)
    oe_ref[...] = s[0] / d[0] + s[1] / d[1]
    or_ref[...] = s[2] / d[2] + s[3] / d[3]


def _normalize(s_raw, deg_raw):
    rb = 1000
    return pl.pallas_call(
        _norm_body,
        grid=(NENT // rb,),
        in_specs=[pl.BlockSpec((4, rb, D), lambda i: (0, i, 0)),
                  pl.BlockSpec((4, rb, 16), lambda i: (0, i, 0))],
        out_specs=[pl.BlockSpec((rb, D), lambda i: (i, 0)),
                   pl.BlockSpec((rb, D), lambda i: (i, 0))],
        out_shape=[jax.ShapeDtypeStruct((NENT, D), jnp.float32),
                   jax.ShapeDtypeStruct((NREL, D), jnp.float32)],
    )(s_raw, deg_raw)


def kernel(e_sender_indices, e_receiver_indices, r_sender_indices,
           r_receiver_indices, E_forward, E_backward, R_forward, R_backward,
           E_message_b, E_gate_b, R_message_b, R_gate_b):
    tables = jnp.stack([R_forward, R_backward, E_forward, E_backward])
    biases = jnp.stack([R_message_b, R_gate_b, E_message_b, E_gate_b])[:, None, :]
    m = _relu_tables(tables, biases).reshape(4 * NENT, D)

    gidx = jnp.stack([r_sender_indices,
                      r_receiver_indices + NENT,
                      e_sender_indices + 2 * NENT,
                      e_receiver_indices + 3 * NENT])
    sidx = jnp.stack([e_receiver_indices, e_sender_indices,
                      r_receiver_indices, r_sender_indices])
    pad = EPS - NEDGE
    gidx = jnp.pad(gidx, ((0, 0), (0, pad))).reshape(4 * NS, NBLK, B)
    sidx = jnp.pad(sidx, ((0, 0), (0, pad)),
                   constant_values=TRASH).reshape(4 * NS, NBLK, B)

    s_raw, deg_raw = _sc_collect()(m, gidx, sidx,
                                   jnp.zeros((ROWS, D), jnp.bfloat16))

    return _normalize(s_raw, deg_raw)


# speedup vs baseline: 6.4008x; 1.0016x over previous
import functools

import jax
import jax.numpy as jnp
from jax import lax
from jax.experimental import pallas as pl
from jax.experimental.pallas import tpu as pltpu
from jax.experimental.pallas import tpu_sc as plsc

NENT = 10000
NREL = 10000
NEDGE = 320000
D = 128
NC = 2
NS = 16
B = 128
NBLK = 160
EPT = NBLK * B
EPS = EPT * NS
ROWS = 10112
RPT = ROWS // NS
NCHUNK = RPT // 8
TRASH = NENT
SCALE = 2048.0


def _sc_body(m_ref, gidx_ref, sidx_ref, zacc_ref,
             s_out_ref, deg_out_ref,
             deg_sh, acc_sh,
             gidx_t, sidx_t, buf0, buf1, buf2, buf3,
             ones_v, zero8_v,
             gsem0, gsem1, gsem2, gsem3, ssem0, ssem1, ssem2, ssem3, dsem):
    bufs = (buf0, buf1, buf2, buf3)
    gsems = (gsem0, gsem1, gsem2, gsem3)
    ssems = (ssem0, ssem1, ssem2, ssem3)
    cid = lax.axis_index("c")
    sid = lax.axis_index("s")
    r0 = sid * RPT

    ones16 = jnp.full((16,), 1.0, jnp.float32)
    zero16 = jnp.zeros((16,), jnp.float32)

    def fill_ones(r, carry):
        ones_v[r, :] = ones16
        return carry

    lax.fori_loop(0, B, fill_ones, 0)
    for r in range(8):
        zero8_v[r, :] = zero16

    for phase in range(2):
        sum_id = phase * 2 + cid
        w = sum_id * NS + sid
        pltpu.sync_copy(gidx_ref.at[w], gidx_t)
        pltpu.sync_copy(sidx_ref.at[w], sidx_t)
        pltpu.sync_copy(zacc_ref.at[pl.ds(r0, RPT)], acc_sh.at[pl.ds(r0, RPT)])

        def zero_deg(c, carry):
            pltpu.sync_copy(zero8_v, deg_sh.at[pl.ds(r0 + c * 8, 8)])
            return carry

        lax.fori_loop(0, NCHUNK, zero_deg, 0)
        plsc.subcore_barrier()

        G = len(bufs)
        for b in range(G):
            pltpu.async_copy(m_ref.at[gidx_t.at[b]], bufs[b], gsems[b])

        def ring(i, carry):
            for b in range(G):
                j = G * i + b
                pltpu.make_async_copy(m_ref.at[gidx_t.at[j]], bufs[b],
                                      gsems[b]).wait()
                pltpu.make_async_copy(bufs[b], acc_sh.at[sidx_t.at[j]],
                                      ssems[b]).start(add=True)
                pltpu.make_async_copy(ones_v, deg_sh.at[sidx_t.at[j]],
                                      dsem).start(add=True)

                @pl.when(j + G < NBLK)
                def _():
                    pltpu.make_async_copy(bufs[b], acc_sh.at[sidx_t.at[j]],
                                          ssems[b]).wait()
                    pltpu.async_copy(m_ref.at[gidx_t.at[j + G]], bufs[b],
                                     gsems[b])
            return carry

        lax.fori_loop(0, NBLK // G, ring, 0)
        for b in range(G):
            pltpu.make_async_copy(bufs[b],
                                  acc_sh.at[sidx_t.at[NBLK - G + b]],
                                  ssems[b]).wait()

        def drain_deg(i, carry):
            pltpu.make_async_copy(ones_v, deg_sh.at[sidx_t.at[0]],
                                  dsem).wait()
            return carry

        lax.fori_loop(0, NBLK, drain_deg, 0)
        plsc.subcore_barrier()

        pltpu.sync_copy(acc_sh.at[pl.ds(r0, RPT)],
                        s_out_ref.at[sum_id, pl.ds(r0, RPT)])
        pltpu.sync_copy(deg_sh.at[pl.ds(r0, RPT)],
                        deg_out_ref.at[sum_id, pl.ds(r0, RPT)])


@functools.lru_cache(maxsize=1)
def _sc_collect():
    mesh = plsc.VectorSubcoreMesh(core_axis_name="c", subcore_axis_name="s",
                                  num_cores=NC, num_subcores=NS)
    return pl.kernel(
        _sc_body,
        out_type=[jax.ShapeDtypeStruct((4, ROWS, D), jnp.int16),
                  jax.ShapeDtypeStruct((4, ROWS, 16), jnp.float32)],
        mesh=mesh,
        compiler_params=pltpu.CompilerParams(use_tc_tiling_on_sc=False),
        scratch_types=[
            pltpu.VMEM_SHARED((ROWS, 16), jnp.float32),
            pltpu.VMEM_SHARED((ROWS, D), jnp.int16),
            pltpu.VMEM((NBLK, B), jnp.int32),
            pltpu.VMEM((NBLK, B), jnp.int32),
            pltpu.VMEM((B, D), jnp.int16),
            pltpu.VMEM((B, D), jnp.int16),
            pltpu.VMEM((B, D), jnp.int16),
            pltpu.VMEM((B, D), jnp.int16),
            pltpu.VMEM((B, 16), jnp.float32),
            pltpu.VMEM((8, 16), jnp.float32),
        ] + [pltpu.SemaphoreType.DMA] * 9,
    )


def _relu_body(t_ref, b_ref, o_ref):
    o_ref[...] = jnp.round(jnp.maximum(t_ref[...] + b_ref[...], 0.0)
                           * SCALE).astype(jnp.int16)


def _relu_tables(tables, biases):
    rb = 1000
    return pl.pallas_call(
        _relu_body,
        grid=(4, NENT // rb),
        in_specs=[pl.BlockSpec((1, rb, D), lambda j, i: (j, i, 0)),
                  pl.BlockSpec((1, 1, D), lambda j, i: (j, 0, 0))],
        out_specs=pl.BlockSpec((1, rb, D), lambda j, i: (j, i, 0)),
        out_shape=jax.ShapeDtypeStruct((4, NENT, D), jnp.int16),
    )(tables, biases)


def _norm_body(s_ref, d_ref, oe_ref, or_ref):
    s = s_ref[...].astype(jnp.float32) * (1.0 / SCALE)
    d = jnp.maximum(d_ref[...][:, :, 0:1], 1.0)
    oe_ref[...] = s[0] / d[0] + s[1] / d[1]
    or_ref[...] = s[2] / d[2] + s[3] / d[3]


def _normalize(s_raw, deg_raw):
    rb = 1000
    return pl.pallas_call(
        _norm_body,
        grid=(NENT // rb,),
        in_specs=[pl.BlockSpec((4, rb, D), lambda i: (0, i, 0)),
                  pl.BlockSpec((4, rb, 16), lambda i: (0, i, 0))],
        out_specs=[pl.BlockSpec((rb, D), lambda i: (i, 0)),
                   pl.BlockSpec((rb, D), lambda i: (i, 0))],
        out_shape=[jax.ShapeDtypeStruct((NENT, D), jnp.float32),
                   jax.ShapeDtypeStruct((NREL, D), jnp.float32)],
    )(s_raw, deg_raw)


def kernel(e_sender_indices, e_receiver_indices, r_sender_indices,
           r_receiver_indices, E_forward, E_backward, R_forward, R_backward,
           E_message_b, E_gate_b, R_message_b, R_gate_b):
    tables = jnp.stack([R_forward, R_backward, E_forward, E_backward])
    biases = jnp.stack([R_message_b, R_gate_b, E_message_b, E_gate_b])[:, None, :]
    m = _relu_tables(tables, biases).reshape(4 * NENT, D)

    gidx = jnp.stack([r_sender_indices,
                      r_receiver_indices + NENT,
                      e_sender_indices + 2 * NENT,
                      e_receiver_indices + 3 * NENT])
    sidx = jnp.stack([e_receiver_indices, e_sender_indices,
                      r_receiver_indices, r_sender_indices])
    pad = EPS - NEDGE
    gidx = jnp.pad(gidx, ((0, 0), (0, pad))).reshape(4 * NS, NBLK, B)
    sidx = jnp.pad(sidx, ((0, 0), (0, pad)),
                   constant_values=TRASH).reshape(4 * NS, NBLK, B)

    s_raw, deg_raw = _sc_collect()(m, gidx, sidx,
                                   jnp.zeros((ROWS, D), jnp.int16))

    return _normalize(s_raw, deg_raw)
